# baseline (speedup 1.0000x reference)
import functools

import jax
import jax.numpy as jnp
from jax.experimental import pallas as pl
from jax.experimental.pallas import tpu as pltpu

B, CIN, HID, LAT, K = 4096, 768, 1024, 32, 512
TILE = 2048
GRID = B // TILE


def _dot(a, b, precision=None):
    return jax.lax.dot_general(
        a, b, (((1,), (0,)), ((), ())),
        preferred_element_type=jnp.float32,
        precision=precision)


def _dott(a, w, precision=None):
    return jax.lax.dot_general(
        a, w, (((1,), (1,)), ((), ())),
        preferred_element_type=jnp.float32,
        precision=precision)


def _vqvae_kernel(x_ref, emb_ref,
                  w1_ref, b1_ref, w2_ref, b2_ref, w3_ref, b3_ref,
                  w4_ref, b4_ref, w5_ref, b5_ref, w6_ref, b6_ref,
                  recon_ref, q_ref, loss_ref):
    i = pl.program_id(0)

    h1 = jax.nn.relu(_dott(x_ref[...], w1_ref[...]) + b1_ref[...])
    h2 = jax.nn.relu(_dott(h1, w2_ref[...]) + b2_ref[...])
    f = jax.nn.relu(_dott(h2, w3_ref[...]) + b3_ref[...])

    emb = emb_ref[...]
    esq = _dott(jnp.ones((1, LAT), jnp.float32), emb * emb,
                precision=jax.lax.Precision.HIGHEST)
    sc = _dott(f, emb, precision=jax.lax.Precision.HIGHEST)
    score = esq - (sc + sc)

    iota = jax.lax.broadcasted_iota(jnp.int32, (TILE, K), 1)

    def first_min_idx(s):
        m = jnp.min(s, axis=1, keepdims=True)
        return jnp.min(jnp.where(s <= m, iota, K), axis=1)[:, None]

    def exact_dist(ec):
        t = f - ec
        t = t * t
        acc = None
        for a in range(4):
            v = t[:, 8 * a:8 * a + 8]
            v = v[:, 0:4] + v[:, 4:8]
            v = v[:, 0:2] + v[:, 2:4]
            v = v[:, 0:1] + v[:, 1:2]
            acc = v if acc is None else acc + v
        return acc

    cands = []
    s_cur = score
    for _ in range(3):
        i_c = first_min_idx(s_cur)
        oh = (iota == i_c).astype(jnp.float32)
        e_c = _dot(oh, emb, precision=jax.lax.Precision.HIGHEST)
        cands.append((exact_dist(e_c), i_c, e_c))
        s_cur = jnp.where(iota == i_c, jnp.inf, s_cur)

    d_w, i_w, e_w = cands[0]
    for d_c, i_c, e_c in cands[1:]:
        take = (d_c < d_w) | ((d_c == d_w) & (i_c < i_w))
        d_w = jnp.where(take, d_c, d_w)
        i_w = jnp.where(take, i_c, i_w)
        e_w = jnp.where(take, e_c, e_w)
    q = e_w

    dq = q - f
    part = jnp.sum(dq * dq)
    q = f + dq

    @pl.when(i == 0)
    def _():
        loss_ref[0, 0] = 0.0
    loss_ref[0, 0] += part
    @pl.when(i == GRID - 1)
    def _():
        loss_ref[0, 0] *= 1.25 / (B * LAT)

    d1 = jax.nn.relu(_dott(q, w4_ref[...]) + b4_ref[...])
    d2 = jax.nn.relu(_dott(d1, w5_ref[...]) + b5_ref[...])
    recon_ref[...] = jax.nn.sigmoid(_dott(d2, w6_ref[...]) + b6_ref[...])
    q_ref[...] = q


@functools.partial(jax.jit, static_argnames=("interpret",))
def kernel(x, emb, enc1_w, enc1_b, enc2_w, enc2_b, enc3_w, enc3_b,
           dec1_w, dec1_b, dec2_w, dec2_b, dec3_w, dec3_b, interpret=False):
    rep = lambda shape: pl.BlockSpec(shape, lambda i: (0,) * len(shape))
    w_specs = []
    ws = []
    for w, b in ((enc1_w, enc1_b), (enc2_w, enc2_b), (enc3_w, enc3_b),
                 (dec1_w, dec1_b), (dec2_w, dec2_b), (dec3_w, dec3_b)):
        ws += [w, b.reshape(1, -1)]
        w_specs += [rep(w.shape), rep((1, b.shape[0]))]

    recon, q, loss = pl.pallas_call(
        _vqvae_kernel,
        grid=(GRID,),
        in_specs=[pl.BlockSpec((TILE, CIN), lambda i: (i, 0)),
                  rep((K, LAT))] + w_specs,
        out_specs=[pl.BlockSpec((TILE, CIN), lambda i: (i, 0)),
                   pl.BlockSpec((TILE, LAT), lambda i: (i, 0)),
                   pl.BlockSpec(memory_space=pltpu.SMEM)],
        out_shape=[jax.ShapeDtypeStruct((B, CIN), jnp.float32),
                   jax.ShapeDtypeStruct((B, LAT), jnp.float32),
                   jax.ShapeDtypeStruct((1, 1), jnp.float32)],
        compiler_params=pltpu.CompilerParams(
            vmem_limit_bytes=100 * 1024 * 1024),
        interpret=interpret,
    )(x, emb, *ws)

    return (recon, loss[0, 0], q)


# speedup vs baseline: 1.3553x; 1.3553x over previous
import functools

import jax
import jax.numpy as jnp
from jax.experimental import pallas as pl
from jax.experimental.pallas import tpu as pltpu

B, CIN, HID, LAT, K = 4096, 768, 1024, 32, 512
TILE = 1024
GRID = B // TILE


def _dot(a, b, precision=None):
    return jax.lax.dot_general(
        a, b, (((1,), (0,)), ((), ())),
        preferred_element_type=jnp.float32,
        precision=precision)


def _dott(a, w, precision=None):
    return jax.lax.dot_general(
        a, w, (((1,), (1,)), ((), ())),
        preferred_element_type=jnp.float32,
        precision=precision)


def _vqvae_kernel(x_ref, emb_ref,
                  w1_ref, b1_ref, w2_ref, b2_ref, w3_ref, b3_ref,
                  w4_ref, b4_ref, w5_ref, b5_ref, w6_ref, b6_ref,
                  recon_ref, q_ref, loss_ref):
    i = pl.program_id(0)

    h1 = jax.nn.relu(_dott(x_ref[...], w1_ref[...]) + b1_ref[...])
    h2 = jax.nn.relu(_dott(h1, w2_ref[...]) + b2_ref[...])
    f = jax.nn.relu(_dott(h2, w3_ref[...]) + b3_ref[...])

    emb = emb_ref[...]
    esq = _dott(jnp.ones((1, LAT), jnp.float32), emb * emb,
                precision=jax.lax.Precision.HIGHEST)
    sc = _dott(f, emb, precision=jax.lax.Precision.HIGHEST)
    score = esq - (sc + sc)

    iota = jax.lax.broadcasted_iota(jnp.int32, (TILE, K), 1)

    def first_min_idx(s):
        m = jnp.min(s, axis=1, keepdims=True)
        return jnp.min(jnp.where(s <= m, iota, K), axis=1)[:, None]

    def exact_dist(ec):
        t = f - ec
        t = t * t
        acc = None
        for a in range(4):
            v = t[:, 8 * a:8 * a + 8]
            v = v[:, 0:4] + v[:, 4:8]
            v = v[:, 0:2] + v[:, 2:4]
            v = v[:, 0:1] + v[:, 1:2]
            acc = v if acc is None else acc + v
        return acc

    cands = []
    s_cur = score
    for _ in range(3):
        i_c = first_min_idx(s_cur)
        oh = (iota == i_c).astype(jnp.float32)
        e_c = _dot(oh, emb, precision=jax.lax.Precision.HIGHEST)
        cands.append((exact_dist(e_c), i_c, e_c))
        s_cur = jnp.where(iota == i_c, jnp.inf, s_cur)

    d_w, i_w, e_w = cands[0]
    for d_c, i_c, e_c in cands[1:]:
        take = (d_c < d_w) | ((d_c == d_w) & (i_c < i_w))
        d_w = jnp.where(take, d_c, d_w)
        i_w = jnp.where(take, i_c, i_w)
        e_w = jnp.where(take, e_c, e_w)
    q = e_w

    dq = q - f
    part = jnp.sum(dq * dq)
    q = f + dq

    @pl.when(i == 0)
    def _():
        loss_ref[0, 0] = 0.0
    loss_ref[0, 0] += part
    @pl.when(i == GRID - 1)
    def _():
        loss_ref[0, 0] *= 1.25 / (B * LAT)

    d1 = jax.nn.relu(_dott(q, w4_ref[...]) + b4_ref[...])
    d2 = jax.nn.relu(_dott(d1, w5_ref[...]) + b5_ref[...])
    recon_ref[...] = jax.nn.sigmoid(_dott(d2, w6_ref[...]) + b6_ref[...])
    q_ref[...] = q


@functools.partial(jax.jit, static_argnames=("interpret",))
def kernel(x, emb, enc1_w, enc1_b, enc2_w, enc2_b, enc3_w, enc3_b,
           dec1_w, dec1_b, dec2_w, dec2_b, dec3_w, dec3_b, interpret=False):
    rep = lambda shape: pl.BlockSpec(shape, lambda i: (0,) * len(shape))
    w_specs = []
    ws = []
    for w, b in ((enc1_w, enc1_b), (enc2_w, enc2_b), (enc3_w, enc3_b),
                 (dec1_w, dec1_b), (dec2_w, dec2_b), (dec3_w, dec3_b)):
        ws += [w, b.reshape(1, -1)]
        w_specs += [rep(w.shape), rep((1, b.shape[0]))]

    recon, q, loss = pl.pallas_call(
        _vqvae_kernel,
        grid=(GRID,),
        in_specs=[pl.BlockSpec((TILE, CIN), lambda i: (i, 0)),
                  rep((K, LAT))] + w_specs,
        out_specs=[pl.BlockSpec((TILE, CIN), lambda i: (i, 0)),
                   pl.BlockSpec((TILE, LAT), lambda i: (i, 0)),
                   pl.BlockSpec(memory_space=pltpu.SMEM)],
        out_shape=[jax.ShapeDtypeStruct((B, CIN), jnp.float32),
                   jax.ShapeDtypeStruct((B, LAT), jnp.float32),
                   jax.ShapeDtypeStruct((1, 1), jnp.float32)],
        compiler_params=pltpu.CompilerParams(
            vmem_limit_bytes=100 * 1024 * 1024),
        interpret=interpret,
    )(x, emb, *ws)

    return (recon, loss[0, 0], q)


# speedup vs baseline: 1.3741x; 1.0138x over previous
import functools

import jax
import jax.numpy as jnp
from jax.experimental import pallas as pl
from jax.experimental.pallas import tpu as pltpu

B, CIN, HID, LAT, K = 4096, 768, 1024, 32, 512
TILE = 1024
GRID = B // TILE


def _dot(a, b, precision=None):
    return jax.lax.dot_general(
        a, b, (((1,), (0,)), ((), ())),
        preferred_element_type=jnp.float32,
        precision=precision)


def _dott(a, w, precision=None):
    return jax.lax.dot_general(
        a, w, (((1,), (1,)), ((), ())),
        preferred_element_type=jnp.float32,
        precision=precision)


def _vqvae_kernel(x_ref, emb_ref,
                  w1_ref, b1_ref, w2_ref, b2_ref, w3_ref, b3_ref,
                  w4_ref, b4_ref, w5_ref, b5_ref, w6_ref, b6_ref,
                  recon_ref, q_ref, loss_ref):
    i = pl.program_id(0)

    h1 = jax.nn.relu(_dott(x_ref[...], w1_ref[...]) + b1_ref[...])
    h2 = jax.nn.relu(_dott(h1, w2_ref[...]) + b2_ref[...])
    f = jax.nn.relu(_dott(h2, w3_ref[...]) + b3_ref[...])

    emb = emb_ref[...]
    esq = _dott(jnp.ones((1, LAT), jnp.float32), emb * emb,
                precision=jax.lax.Precision.HIGHEST)
    sc = _dott(f, emb, precision=jax.lax.Precision.HIGHEST)
    score = esq - (sc + sc)

    iota = jax.lax.broadcasted_iota(jnp.int32, (TILE, K), 1)

    mrow = jnp.min(score, axis=1, keepdims=True)
    spos = score - mrow
    key = (jax.lax.bitcast_convert_type(spos, jnp.int32) & ~511) | iota

    def exact_dist(ec):
        t = f - ec
        t = t * t
        acc = None
        for a in range(4):
            v = t[:, 8 * a:8 * a + 8]
            v = v[:, 0:4] + v[:, 4:8]
            v = v[:, 0:2] + v[:, 2:4]
            v = v[:, 0:1] + v[:, 1:2]
            acc = v if acc is None else acc + v
        return acc

    cands = []
    k_cur = key
    for _ in range(3):
        k_min = jnp.min(k_cur, axis=1, keepdims=True)
        i_c = k_min & 511
        oh = (iota == i_c).astype(jnp.float32)
        e_c = _dot(oh, emb, precision=jax.lax.Precision.HIGHEST)
        cands.append((exact_dist(e_c), i_c, e_c))
        k_cur = jnp.where(k_cur == k_min, jnp.int32(0x7FFFFFFF), k_cur)

    d_w, i_w, e_w = cands[0]
    for d_c, i_c, e_c in cands[1:]:
        take = (d_c < d_w) | ((d_c == d_w) & (i_c < i_w))
        d_w = jnp.where(take, d_c, d_w)
        i_w = jnp.where(take, i_c, i_w)
        e_w = jnp.where(take, e_c, e_w)
    q = e_w

    dq = q - f
    part = jnp.sum(dq * dq)
    q = f + dq

    @pl.when(i == 0)
    def _():
        loss_ref[0, 0] = 0.0
    loss_ref[0, 0] += part
    @pl.when(i == GRID - 1)
    def _():
        loss_ref[0, 0] *= 1.25 / (B * LAT)

    d1 = jax.nn.relu(_dott(q, w4_ref[...]) + b4_ref[...])
    d2 = jax.nn.relu(_dott(d1, w5_ref[...]) + b5_ref[...])
    recon_ref[...] = jax.nn.sigmoid(_dott(d2, w6_ref[...]) + b6_ref[...])
    q_ref[...] = q


@functools.partial(jax.jit, static_argnames=("interpret",))
def kernel(x, emb, enc1_w, enc1_b, enc2_w, enc2_b, enc3_w, enc3_b,
           dec1_w, dec1_b, dec2_w, dec2_b, dec3_w, dec3_b, interpret=False):
    rep = lambda shape: pl.BlockSpec(shape, lambda i: (0,) * len(shape))
    w_specs = []
    ws = []
    for w, b in ((enc1_w, enc1_b), (enc2_w, enc2_b), (enc3_w, enc3_b),
                 (dec1_w, dec1_b), (dec2_w, dec2_b), (dec3_w, dec3_b)):
        ws += [w, b.reshape(1, -1)]
        w_specs += [rep(w.shape), rep((1, b.shape[0]))]

    recon, q, loss = pl.pallas_call(
        _vqvae_kernel,
        grid=(GRID,),
        in_specs=[pl.BlockSpec((TILE, CIN), lambda i: (i, 0)),
                  rep((K, LAT))] + w_specs,
        out_specs=[pl.BlockSpec((TILE, CIN), lambda i: (i, 0)),
                   pl.BlockSpec((TILE, LAT), lambda i: (i, 0)),
                   pl.BlockSpec(memory_space=pltpu.SMEM)],
        out_shape=[jax.ShapeDtypeStruct((B, CIN), jnp.float32),
                   jax.ShapeDtypeStruct((B, LAT), jnp.float32),
                   jax.ShapeDtypeStruct((1, 1), jnp.float32)],
        compiler_params=pltpu.CompilerParams(
            vmem_limit_bytes=100 * 1024 * 1024),
        interpret=interpret,
    )(x, emb, *ws)

    return (recon, loss[0, 0], q)
